# baseline (speedup 1.0000x reference)
import functools

import jax
import jax.numpy as jnp
import numpy as np
from jax import lax
from jax.experimental import pallas as pl
from jax.experimental.pallas import tpu as pltpu
from jax.experimental.pallas import tpu_sc as plsc

N = 10000
E = 320000
D = 128
DE = 16

NC = 2
NS = 16
NW = NC * NS
EPW = E // NW
G = 80
KCH = EPW // G
CHO = 80
NCH = N // CHO


def _edge_linear_body(ea_ref, we_ref, be_ref, ee_ref):
    ee = (jnp.dot(ea_ref[...], we_ref[...],
                  preferred_element_type=jnp.float32)
          + be_ref[...])
    ab = lax.bitcast_convert_type(ee[:, :64], jnp.uint32)
    bb = lax.bitcast_convert_type(ee[:, 64:], jnp.uint32)
    ar = ab + 0x7FFF + ((ab >> 16) & 1)
    br = bb + 0x7FFF + ((bb >> 16) & 1)
    packed = (ar >> 16) | (br & jnp.uint32(0xFFFF0000))
    ee_ref[...] = lax.bitcast_convert_type(packed, jnp.int32)


def _edge_linear(edge_attr, We, be):
    BE = 6400
    return pl.pallas_call(
        _edge_linear_body,
        grid=(E // BE,),
        in_specs=[
            pl.BlockSpec((BE, DE), lambda i: (i, 0)),
            pl.BlockSpec((DE, D), lambda i: (0, 0)),
            pl.BlockSpec((1, D), lambda i: (0, 0)),
        ],
        out_specs=pl.BlockSpec((BE, D // 2), lambda i: (i, 0)),
        out_shape=jax.ShapeDtypeStruct((E, D // 2), jnp.int32),
    )(edge_attr, We, be.reshape(1, D))


def _edge_stage_body(x_hbm, ee_hbm, src_hbm, dst_hbm, out_hbm,
                     src_idx, dst_idx, rows, eebuf, agg_sh,
                     sg0, sg1, se0, se1, ss0, ss1, si0, si1, si2, si3):
    cid = lax.axis_index("c")
    sid = lax.axis_index("s")
    wid = sid * NC + cid
    base_w = wid * EPW

    sem_g = [sg0, sg1]
    sem_e = [se0, se1]
    sem_s = [ss0, ss1]
    sem_i = [si0, si1, si2, si3]

    zv = jnp.zeros((16,), jnp.float32)

    @pl.loop(0, G)
    def _zero_fill(r):
        for cc in range(D // 16):
            rows[0, r, pl.ds(cc * 16, 16)] = zv

    @pl.loop(sid, NCH, step=NS)
    def _zero_out(j):
        pltpu.sync_copy(rows.at[0], agg_sh.at[pl.ds(j * CHO, CHO)])

    plsc.subcore_barrier()

    def issue_idx(k, islot):
        base = base_w + k * G
        pltpu.async_copy(src_hbm.at[pl.ds(base, G)], src_idx.at[islot],
                         sem_i[islot])
        pltpu.async_copy(dst_hbm.at[pl.ds(base, G)], dst_idx.at[islot],
                         sem_i[islot])

    def wait_idx(islot):
        pltpu.make_async_copy(src_hbm.at[pl.ds(0, G)], src_idx.at[islot],
                              sem_i[islot]).wait()
        pltpu.make_async_copy(dst_hbm.at[pl.ds(0, G)], dst_idx.at[islot],
                              sem_i[islot]).wait()

    def issue_gather_ee(k, b, islot):
        base = base_w + k * G
        pltpu.async_copy(x_hbm.at[src_idx.at[islot]], rows.at[b], sem_g[b])
        pltpu.async_copy(ee_hbm.at[pl.ds(base, G)], eebuf.at[b], sem_e[b])

    def phase(k, p):
        b = p % 2
        b1 = (p + 1) % 2
        pltpu.make_async_copy(x_hbm.at[pl.ds(0, G), :], rows.at[b],
                              sem_g[b]).wait()
        pltpu.make_async_copy(ee_hbm.at[pl.ds(0, G), :], eebuf.at[b],
                              sem_e[b]).wait()

        @pl.when(k + 1 < KCH)
        def _():
            @pl.when(k >= 1)
            def _():
                pltpu.make_async_copy(rows.at[b1], agg_sh.at[pl.ds(0, G)],
                                      sem_s[b1]).wait()
            wait_idx((p + 1) % 4)
            issue_gather_ee(k + 1, b1, (p + 1) % 4)

        @pl.when(k + 2 < KCH)
        def _():
            issue_idx(k + 2, (p + 2) % 4)

        @pl.loop(0, G)
        def _relu_add(r):
            for j in range(D // 32):
                c = j * 16
                ev = eebuf[b, r, pl.ds(c, 16)]
                lo = lax.bitcast_convert_type(ev << 16, jnp.float32)
                hi = lax.bitcast_convert_type(ev & jnp.int32(-65536),
                                              jnp.float32)
                v1 = rows[b, r, pl.ds(c, 16)] + lo
                v2 = rows[b, r, pl.ds(64 + c, 16)] + hi
                rows[b, r, pl.ds(c, 16)] = jnp.maximum(v1, 0.0)
                rows[b, r, pl.ds(64 + c, 16)] = jnp.maximum(v2, 0.0)

        pltpu.async_copy(rows.at[b], agg_sh.at[dst_idx.at[p % 4]], sem_s[b],
                         add=True)

    issue_idx(0, 0)
    issue_idx(1, 1)
    wait_idx(0)
    issue_gather_ee(0, 0, 0)

    @pl.loop(0, KCH - 1, step=4)
    def _main(k0):
        for p in range(4):
            phase(k0 + p, p)

    phase(KCH - 1, (KCH - 1) % 4)

    pltpu.make_async_copy(rows.at[0], agg_sh.at[pl.ds(0, G)], sem_s[0]).wait()
    pltpu.make_async_copy(rows.at[1], agg_sh.at[pl.ds(0, G)], sem_s[1]).wait()

    plsc.subcore_barrier()

    @pl.loop(sid, NCH, step=NS)
    def _copy_out(j):
        r0 = j * CHO
        pltpu.sync_copy(agg_sh.at[pl.ds(r0, CHO)],
                        out_hbm.at[cid, pl.ds(r0, CHO), :])


def _edge_stage(x, ee, src, dst):
    mesh = plsc.VectorSubcoreMesh(core_axis_name="c", subcore_axis_name="s",
                                  num_cores=NC, num_subcores=NS)
    f = pl.kernel(
        _edge_stage_body,
        out_type=jax.ShapeDtypeStruct((NC, N, D), jnp.float32),
        mesh=mesh,
        compiler_params=pltpu.CompilerParams(needs_layout_passes=False),
        scratch_types=[
            pltpu.VMEM((4, G), jnp.int32),
            pltpu.VMEM((4, G), jnp.int32),
            pltpu.VMEM((2, G, D), jnp.float32),
            pltpu.VMEM((2, G, D // 2), jnp.int32),
            pltpu.VMEM_SHARED((N, D), jnp.float32),
        ] + [pltpu.SemaphoreType.DMA] * 10,
    )
    return f(x, ee, src, dst)


def _node_stage_body(x_ref, agg_ref, eps_ref, w1_ref, b1_ref, w2_ref, b2_ref,
                     g_ref, bt_ref, out_ref):
    x = x_ref[...]
    h = (1.0 + eps_ref[0, 0]) * x + agg_ref[0] + agg_ref[1]
    t = jnp.maximum(jnp.dot(h, w1_ref[...],
                            preferred_element_type=jnp.float32) + b1_ref[...],
                    0.0)
    o = jnp.dot(t, w2_ref[...],
                preferred_element_type=jnp.float32) + b2_ref[...]
    mu = jnp.mean(o, axis=-1, keepdims=True)
    var = jnp.mean(jnp.square(o - mu), axis=-1, keepdims=True)
    o = (o - mu) * lax.rsqrt(var + 1e-5) * g_ref[...] + bt_ref[...]
    out_ref[...] = jnp.maximum(o, 0.0)


def _node_stage(x, agg, eps, W1, b1, W2, b2, g, bt):
    BN = 2000
    grid = (N // BN,)
    return pl.pallas_call(
        _node_stage_body,
        grid=grid,
        in_specs=[
            pl.BlockSpec((BN, D), lambda i: (i, 0)),
            pl.BlockSpec((NC, BN, D), lambda i: (0, i, 0)),
            pl.BlockSpec((1, 1), lambda i: (0, 0)),
            pl.BlockSpec((D, D), lambda i: (0, 0)),
            pl.BlockSpec((1, D), lambda i: (0, 0)),
            pl.BlockSpec((D, D), lambda i: (0, 0)),
            pl.BlockSpec((1, D), lambda i: (0, 0)),
            pl.BlockSpec((1, D), lambda i: (0, 0)),
            pl.BlockSpec((1, D), lambda i: (0, 0)),
        ],
        out_specs=pl.BlockSpec((BN, D), lambda i: (i, 0)),
        out_shape=jax.ShapeDtypeStruct((N, D), jnp.float32),
    )(x, agg, eps.reshape(1, 1), W1, b1.reshape(1, D), W2, b2.reshape(1, D),
      g.reshape(1, D), bt.reshape(1, D))


def kernel(x, edge_index, edge_attr,
           We0, be0, W10, b10, W20, b20, eps0, g0, bt0,
           We1, be1, W11, b11, W21, b21, eps1, g1, bt1):
    src = edge_index[0].astype(jnp.int32)
    dst = edge_index[1].astype(jnp.int32)

    ee0 = _edge_linear(edge_attr, We0, be0)
    ee1 = _edge_linear(edge_attr, We1, be1)

    agg0 = _edge_stage(x, ee0, src, dst)
    h1 = _node_stage(x, agg0, eps0, W10, b10, W20, b20, g0, bt0)

    agg1 = _edge_stage(h1, ee1, src, dst)
    h2 = _node_stage(h1, agg1, eps1, W11, b11, W21, b21, g1, bt1)
    return h2


# speedup vs baseline: 1.0220x; 1.0220x over previous
import functools

import jax
import jax.numpy as jnp
import numpy as np
from jax import lax
from jax.experimental import pallas as pl
from jax.experimental.pallas import tpu as pltpu
from jax.experimental.pallas import tpu_sc as plsc

N = 10000
E = 320000
D = 128
DE = 16

NC = 2
NS = 16
NW = NC * NS
EPW = E // NW
G = 80
KCH = EPW // G
CHO = 80
NCH = N // CHO


def _pack_bf16_pair(v):
    ab = lax.bitcast_convert_type(v[:, :64], jnp.uint32)
    bb = lax.bitcast_convert_type(v[:, 64:], jnp.uint32)
    ar = ab + 0x7FFF + ((ab >> 16) & 1)
    br = bb + 0x7FFF + ((bb >> 16) & 1)
    packed = (ar >> 16) | (br & jnp.uint32(0xFFFF0000))
    return lax.bitcast_convert_type(packed, jnp.int32)


def _edge_linear_body(ea_ref, we_ref, be_ref, ee0_ref, ee1_ref):
    ee = (jnp.dot(ea_ref[...], we_ref[...],
                  preferred_element_type=jnp.float32)
          + be_ref[...])
    ee0_ref[...] = _pack_bf16_pair(ee[:, :D])
    ee1_ref[...] = _pack_bf16_pair(ee[:, D:])


def _edge_linear2(edge_attr, We0, be0, We1, be1):
    BE = 6400
    We = jnp.concatenate([We0, We1], axis=1)
    be = jnp.concatenate([be0, be1]).reshape(1, 2 * D)
    return pl.pallas_call(
        _edge_linear_body,
        grid=(E // BE,),
        in_specs=[
            pl.BlockSpec((BE, DE), lambda i: (i, 0)),
            pl.BlockSpec((DE, 2 * D), lambda i: (0, 0)),
            pl.BlockSpec((1, 2 * D), lambda i: (0, 0)),
        ],
        out_specs=[pl.BlockSpec((BE, D // 2), lambda i: (i, 0)),
                   pl.BlockSpec((BE, D // 2), lambda i: (i, 0))],
        out_shape=[jax.ShapeDtypeStruct((E, D // 2), jnp.int32),
                   jax.ShapeDtypeStruct((E, D // 2), jnp.int32)],
    )(edge_attr, We, be)


def _edge_stage_body(x_hbm, ee_hbm, src_hbm, dst_hbm, out_hbm,
                     src_idx, dst_idx, rows, eebuf, agg_sh,
                     sg0, sg1, se0, se1, ss0, ss1, si0, si1, si2, si3):
    cid = lax.axis_index("c")
    sid = lax.axis_index("s")
    wid = sid * NC + cid
    base_w = wid * EPW

    sem_g = [sg0, sg1]
    sem_e = [se0, se1]
    sem_s = [ss0, ss1]
    sem_i = [si0, si1, si2, si3]

    zv = jnp.zeros((16,), jnp.float32)

    @pl.loop(0, G)
    def _zero_fill(r):
        for cc in range(D // 16):
            rows[0, r, pl.ds(cc * 16, 16)] = zv

    @pl.loop(sid, NCH, step=NS)
    def _zero_out(j):
        pltpu.sync_copy(rows.at[0, pl.ds(0, CHO)],
                        agg_sh.at[pl.ds(j * CHO, CHO)])

    plsc.subcore_barrier()

    def issue_idx(k, islot):
        base = base_w + k * G
        pltpu.async_copy(src_hbm.at[pl.ds(base, G)], src_idx.at[islot],
                         sem_i[islot])
        pltpu.async_copy(dst_hbm.at[pl.ds(base, G)], dst_idx.at[islot],
                         sem_i[islot])

    def wait_idx(islot):
        pltpu.make_async_copy(src_hbm.at[pl.ds(0, G)], src_idx.at[islot],
                              sem_i[islot]).wait()
        pltpu.make_async_copy(dst_hbm.at[pl.ds(0, G)], dst_idx.at[islot],
                              sem_i[islot]).wait()

    def issue_gather_ee(k, b, islot):
        base = base_w + k * G
        pltpu.async_copy(x_hbm.at[src_idx.at[islot]], rows.at[b], sem_g[b])
        pltpu.async_copy(ee_hbm.at[pl.ds(base, G)], eebuf.at[b], sem_e[b])

    def phase(k, p):
        b = p % 2
        b1 = (p + 1) % 2
        pltpu.make_async_copy(x_hbm.at[pl.ds(0, G), :], rows.at[b],
                              sem_g[b]).wait()
        pltpu.make_async_copy(ee_hbm.at[pl.ds(0, G), :], eebuf.at[b],
                              sem_e[b]).wait()

        @pl.when(k + 1 < KCH)
        def _():
            @pl.when(k >= 1)
            def _():
                pltpu.make_async_copy(rows.at[b1], agg_sh.at[pl.ds(0, G)],
                                      sem_s[b1]).wait()
            wait_idx((p + 1) % 4)
            issue_gather_ee(k + 1, b1, (p + 1) % 4)

        @pl.when(k + 2 < KCH)
        def _():
            issue_idx(k + 2, (p + 2) % 4)

        @pl.loop(0, G)
        def _relu_add(r):
            for j in range(D // 32):
                c = j * 16
                ev = eebuf[b, r, pl.ds(c, 16)]
                lo = lax.bitcast_convert_type(ev << 16, jnp.float32)
                hi = lax.bitcast_convert_type(ev & jnp.int32(-65536),
                                              jnp.float32)
                v1 = rows[b, r, pl.ds(c, 16)] + lo
                v2 = rows[b, r, pl.ds(64 + c, 16)] + hi
                rows[b, r, pl.ds(c, 16)] = jnp.maximum(v1, 0.0)
                rows[b, r, pl.ds(64 + c, 16)] = jnp.maximum(v2, 0.0)

        pltpu.async_copy(rows.at[b], agg_sh.at[dst_idx.at[p % 4]], sem_s[b],
                         add=True)

    issue_idx(0, 0)
    issue_idx(1, 1)
    wait_idx(0)
    issue_gather_ee(0, 0, 0)

    @pl.loop(0, KCH - 1, step=4)
    def _main(k0):
        for p in range(4):
            phase(k0 + p, p)

    phase(KCH - 1, (KCH - 1) % 4)

    pltpu.make_async_copy(rows.at[0], agg_sh.at[pl.ds(0, G)], sem_s[0]).wait()
    pltpu.make_async_copy(rows.at[1], agg_sh.at[pl.ds(0, G)], sem_s[1]).wait()

    plsc.subcore_barrier()

    @pl.loop(sid, NCH, step=NS)
    def _copy_out(j):
        r0 = j * CHO
        pltpu.sync_copy(agg_sh.at[pl.ds(r0, CHO)],
                        out_hbm.at[cid, pl.ds(r0, CHO), :])


def _edge_stage(x, ee, src, dst):
    mesh = plsc.VectorSubcoreMesh(core_axis_name="c", subcore_axis_name="s",
                                  num_cores=NC, num_subcores=NS)
    f = pl.kernel(
        _edge_stage_body,
        out_type=jax.ShapeDtypeStruct((NC, N, D), jnp.float32),
        mesh=mesh,
        compiler_params=pltpu.CompilerParams(needs_layout_passes=False),
        scratch_types=[
            pltpu.VMEM((4, G), jnp.int32),
            pltpu.VMEM((4, G), jnp.int32),
            pltpu.VMEM((2, G, D), jnp.float32),
            pltpu.VMEM((2, G, D // 2), jnp.int32),
            pltpu.VMEM_SHARED((N, D), jnp.float32),
        ] + [pltpu.SemaphoreType.DMA] * 10,
    )
    return f(x, ee, src, dst)


def _node_stage_body(x_ref, agg_ref, eps_ref, w1_ref, b1_ref, w2_ref, b2_ref,
                     g_ref, bt_ref, out_ref):
    x = x_ref[...]
    h = (1.0 + eps_ref[0, 0]) * x + agg_ref[0] + agg_ref[1]
    t = jnp.maximum(jnp.dot(h, w1_ref[...],
                            preferred_element_type=jnp.float32) + b1_ref[...],
                    0.0)
    o = jnp.dot(t, w2_ref[...],
                preferred_element_type=jnp.float32) + b2_ref[...]
    mu = jnp.mean(o, axis=-1, keepdims=True)
    var = jnp.mean(jnp.square(o - mu), axis=-1, keepdims=True)
    o = (o - mu) * lax.rsqrt(var + 1e-5) * g_ref[...] + bt_ref[...]
    out_ref[...] = jnp.maximum(o, 0.0)


def _node_stage(x, agg, eps, W1, b1, W2, b2, g, bt):
    BN = 2000
    grid = (N // BN,)
    return pl.pallas_call(
        _node_stage_body,
        grid=grid,
        in_specs=[
            pl.BlockSpec((BN, D), lambda i: (i, 0)),
            pl.BlockSpec((NC, BN, D), lambda i: (0, i, 0)),
            pl.BlockSpec((1, 1), lambda i: (0, 0)),
            pl.BlockSpec((D, D), lambda i: (0, 0)),
            pl.BlockSpec((1, D), lambda i: (0, 0)),
            pl.BlockSpec((D, D), lambda i: (0, 0)),
            pl.BlockSpec((1, D), lambda i: (0, 0)),
            pl.BlockSpec((1, D), lambda i: (0, 0)),
            pl.BlockSpec((1, D), lambda i: (0, 0)),
        ],
        out_specs=pl.BlockSpec((BN, D), lambda i: (i, 0)),
        out_shape=jax.ShapeDtypeStruct((N, D), jnp.float32),
    )(x, agg, eps.reshape(1, 1), W1, b1.reshape(1, D), W2, b2.reshape(1, D),
      g.reshape(1, D), bt.reshape(1, D))


def kernel(x, edge_index, edge_attr,
           We0, be0, W10, b10, W20, b20, eps0, g0, bt0,
           We1, be1, W11, b11, W21, b21, eps1, g1, bt1):
    src = edge_index[0].astype(jnp.int32)
    dst = edge_index[1].astype(jnp.int32)

    ee0, ee1 = _edge_linear2(edge_attr, We0, be0, We1, be1)

    agg0 = _edge_stage(x, ee0, src, dst)
    h1 = _node_stage(x, agg0, eps0, W10, b10, W20, b20, g0, bt0)

    agg1 = _edge_stage(h1, ee1, src, dst)
    h2 = _node_stage(h1, agg1, eps1, W11, b11, W21, b21, g1, bt1)
    return h2


# speedup vs baseline: 1.1156x; 1.0916x over previous
import functools

import jax
import jax.numpy as jnp
import numpy as np
from jax import lax
from jax.experimental import pallas as pl
from jax.experimental.pallas import tpu as pltpu
from jax.experimental.pallas import tpu_sc as plsc

N = 10000
E = 320000
D = 128
DE = 16

NC = 2
NS = 16
NW = NC * NS
EPW = E // NW
G = 40
KCH = EPW // G
CHO = 40
NCH = N // CHO


def _pack_bf16_pair(v):
    ab = lax.bitcast_convert_type(v[:, :64], jnp.uint32)
    bb = lax.bitcast_convert_type(v[:, 64:], jnp.uint32)
    ar = ab + 0x7FFF + ((ab >> 16) & 1)
    br = bb + 0x7FFF + ((bb >> 16) & 1)
    packed = (ar >> 16) | (br & jnp.uint32(0xFFFF0000))
    return lax.bitcast_convert_type(packed, jnp.int32)


def _edge_linear_body(ea_ref, we_ref, be_ref, ee0_ref, ee1_ref):
    ee = (jnp.dot(ea_ref[...], we_ref[...],
                  preferred_element_type=jnp.float32)
          + be_ref[...])
    ee0_ref[...] = _pack_bf16_pair(ee[:, :D])
    ee1_ref[...] = _pack_bf16_pair(ee[:, D:])


def _edge_linear2(edge_attr, We0, be0, We1, be1):
    BE = 6400
    We = jnp.concatenate([We0, We1], axis=1)
    be = jnp.concatenate([be0, be1]).reshape(1, 2 * D)
    return pl.pallas_call(
        _edge_linear_body,
        grid=(E // BE,),
        in_specs=[
            pl.BlockSpec((BE, DE), lambda i: (i, 0)),
            pl.BlockSpec((DE, 2 * D), lambda i: (0, 0)),
            pl.BlockSpec((1, 2 * D), lambda i: (0, 0)),
        ],
        out_specs=[pl.BlockSpec((BE, D // 2), lambda i: (i, 0)),
                   pl.BlockSpec((BE, D // 2), lambda i: (i, 0))],
        out_shape=[jax.ShapeDtypeStruct((E, D // 2), jnp.int32),
                   jax.ShapeDtypeStruct((E, D // 2), jnp.int32)],
    )(edge_attr, We, be)


def _edge_stage_body(x_hbm, ee_hbm, src_hbm, dst_hbm, out_hbm,
                     src_idx, dst_idx, rows, eebuf, agg_sh,
                     sg0, sg1, sg2, sg3, se0, se1, se2, se3,
                     ss0, ss1, ss2, ss3, si0, si1, si2, si3,
                     sd0, sd1, sd2, sd3):
    cid = lax.axis_index("c")
    sid = lax.axis_index("s")
    wid = sid * NC + cid
    base_w = wid * EPW

    sem_g = [sg0, sg1, sg2, sg3]
    sem_e = [se0, se1, se2, se3]
    sem_s = [ss0, ss1, ss2, ss3]
    sem_i = [si0, si1, si2, si3]
    sem_d = [sd0, sd1, sd2, sd3]

    zv = jnp.zeros((16,), jnp.float32)

    @pl.loop(0, G)
    def _zero_fill(r):
        for cc in range(D // 16):
            rows[0, r, pl.ds(cc * 16, 16)] = zv

    @pl.loop(sid, NCH, step=NS)
    def _zero_out(j):
        pltpu.sync_copy(rows.at[0, pl.ds(0, CHO)],
                        agg_sh.at[pl.ds(j * CHO, CHO)])

    plsc.subcore_barrier()

    def issue_src_idx(k, islot):
        base = base_w + k * G
        pltpu.async_copy(src_hbm.at[pl.ds(base, G)], src_idx.at[islot],
                         sem_i[islot])

    def wait_src_idx(islot):
        pltpu.make_async_copy(src_hbm.at[pl.ds(0, G)], src_idx.at[islot],
                              sem_i[islot]).wait()

    def issue_dst_idx(k, islot):
        base = base_w + k * G
        pltpu.async_copy(dst_hbm.at[pl.ds(base, G)], dst_idx.at[islot],
                         sem_d[islot])

    def wait_dst_idx(islot):
        pltpu.make_async_copy(dst_hbm.at[pl.ds(0, G)], dst_idx.at[islot],
                              sem_d[islot]).wait()

    def issue_gather_ee(k, b, islot):
        base = base_w + k * G
        pltpu.async_copy(x_hbm.at[src_idx.at[islot]], rows.at[b], sem_g[b])
        pltpu.async_copy(ee_hbm.at[pl.ds(base, G)], eebuf.at[b], sem_e[b])

    def phase(k, p):
        b = p % 4
        b2 = (p + 2) % 4
        pltpu.make_async_copy(x_hbm.at[pl.ds(0, G), :], rows.at[b],
                              sem_g[b]).wait()
        pltpu.make_async_copy(ee_hbm.at[pl.ds(0, G), :], eebuf.at[b],
                              sem_e[b]).wait()

        @pl.when(k + 2 < KCH)
        def _():
            @pl.when(k >= 2)
            def _():
                pltpu.make_async_copy(rows.at[b2], agg_sh.at[pl.ds(0, G)],
                                      sem_s[b2]).wait()
            issue_dst_idx(k + 2, b2)
            wait_src_idx(b2)
            issue_gather_ee(k + 2, b2, b2)

        @pl.when(k + 3 < KCH)
        def _():
            issue_src_idx(k + 3, (p + 3) % 4)

        @pl.loop(0, G)
        def _relu_add(r):
            for j in range(D // 32):
                c = j * 16
                ev = eebuf[b, r, pl.ds(c, 16)]
                lo = lax.bitcast_convert_type(ev << 16, jnp.float32)
                hi = lax.bitcast_convert_type(ev & jnp.int32(-65536),
                                              jnp.float32)
                v1 = rows[b, r, pl.ds(c, 16)] + lo
                v2 = rows[b, r, pl.ds(64 + c, 16)] + hi
                rows[b, r, pl.ds(c, 16)] = jnp.maximum(v1, 0.0)
                rows[b, r, pl.ds(64 + c, 16)] = jnp.maximum(v2, 0.0)

        wait_dst_idx(b)
        pltpu.async_copy(rows.at[b], agg_sh.at[dst_idx.at[b]], sem_s[b],
                         add=True)

    issue_src_idx(0, 0)
    issue_src_idx(1, 1)
    issue_src_idx(2, 2)
    issue_dst_idx(0, 0)
    issue_dst_idx(1, 1)
    wait_src_idx(0)
    issue_gather_ee(0, 0, 0)
    wait_src_idx(1)
    issue_gather_ee(1, 1, 1)

    @pl.loop(0, KCH - 2, step=4)
    def _main(k0):
        for p in range(4):
            phase(k0 + p, p)

    phase(KCH - 2, (KCH - 2) % 4)
    phase(KCH - 1, (KCH - 1) % 4)

    for q in range(4):
        pltpu.make_async_copy(rows.at[q], agg_sh.at[pl.ds(0, G)],
                              sem_s[q]).wait()

    plsc.subcore_barrier()

    @pl.loop(sid, NCH, step=NS)
    def _copy_out(j):
        r0 = j * CHO
        pltpu.sync_copy(agg_sh.at[pl.ds(r0, CHO)],
                        out_hbm.at[cid, pl.ds(r0, CHO), :])


def _edge_stage(x, ee, src, dst):
    mesh = plsc.VectorSubcoreMesh(core_axis_name="c", subcore_axis_name="s",
                                  num_cores=NC, num_subcores=NS)
    f = pl.kernel(
        _edge_stage_body,
        out_type=jax.ShapeDtypeStruct((NC, N, D), jnp.float32),
        mesh=mesh,
        compiler_params=pltpu.CompilerParams(needs_layout_passes=False),
        scratch_types=[
            pltpu.VMEM((4, G), jnp.int32),
            pltpu.VMEM((4, G), jnp.int32),
            pltpu.VMEM((4, G, D), jnp.float32),
            pltpu.VMEM((4, G, D // 2), jnp.int32),
            pltpu.VMEM_SHARED((N, D), jnp.float32),
        ] + [pltpu.SemaphoreType.DMA] * 20,
    )
    return f(x, ee, src, dst)


def _node_stage_body(x_ref, agg_ref, eps_ref, w1_ref, b1_ref, w2_ref, b2_ref,
                     g_ref, bt_ref, out_ref):
    x = x_ref[...]
    h = (1.0 + eps_ref[0, 0]) * x + agg_ref[0] + agg_ref[1]
    t = jnp.maximum(jnp.dot(h, w1_ref[...],
                            preferred_element_type=jnp.float32) + b1_ref[...],
                    0.0)
    o = jnp.dot(t, w2_ref[...],
                preferred_element_type=jnp.float32) + b2_ref[...]
    mu = jnp.mean(o, axis=-1, keepdims=True)
    var = jnp.mean(jnp.square(o - mu), axis=-1, keepdims=True)
    o = (o - mu) * lax.rsqrt(var + 1e-5) * g_ref[...] + bt_ref[...]
    out_ref[...] = jnp.maximum(o, 0.0)


def _node_stage(x, agg, eps, W1, b1, W2, b2, g, bt):
    BN = 2000
    grid = (N // BN,)
    return pl.pallas_call(
        _node_stage_body,
        grid=grid,
        in_specs=[
            pl.BlockSpec((BN, D), lambda i: (i, 0)),
            pl.BlockSpec((NC, BN, D), lambda i: (0, i, 0)),
            pl.BlockSpec((1, 1), lambda i: (0, 0)),
            pl.BlockSpec((D, D), lambda i: (0, 0)),
            pl.BlockSpec((1, D), lambda i: (0, 0)),
            pl.BlockSpec((D, D), lambda i: (0, 0)),
            pl.BlockSpec((1, D), lambda i: (0, 0)),
            pl.BlockSpec((1, D), lambda i: (0, 0)),
            pl.BlockSpec((1, D), lambda i: (0, 0)),
        ],
        out_specs=pl.BlockSpec((BN, D), lambda i: (i, 0)),
        out_shape=jax.ShapeDtypeStruct((N, D), jnp.float32),
    )(x, agg, eps.reshape(1, 1), W1, b1.reshape(1, D), W2, b2.reshape(1, D),
      g.reshape(1, D), bt.reshape(1, D))


def kernel(x, edge_index, edge_attr,
           We0, be0, W10, b10, W20, b20, eps0, g0, bt0,
           We1, be1, W11, b11, W21, b21, eps1, g1, bt1):
    src = edge_index[0].astype(jnp.int32)
    dst = edge_index[1].astype(jnp.int32)

    ee0, ee1 = _edge_linear2(edge_attr, We0, be0, We1, be1)

    agg0 = _edge_stage(x, ee0, src, dst)
    h1 = _node_stage(x, agg0, eps0, W10, b10, W20, b20, g0, bt0)

    agg1 = _edge_stage(h1, ee1, src, dst)
    h2 = _node_stage(h1, agg1, eps1, W11, b11, W21, b21, g1, bt1)
    return h2


# speedup vs baseline: 1.1315x; 1.0143x over previous
import functools

import jax
import jax.numpy as jnp
import numpy as np
from jax import lax
from jax.experimental import pallas as pl
from jax.experimental.pallas import tpu as pltpu
from jax.experimental.pallas import tpu_sc as plsc

N = 10000
E = 320000
D = 128
DE = 16

NC = 2
NS = 16
NW = NC * NS
EPW = E // NW
G = 40
KCH = EPW // G
CHO = 40
NCH = N // CHO
CHO_OUT = 200
NCH_OUT = N // CHO_OUT


def _pack_bf16_pair(v):
    ab = lax.bitcast_convert_type(v[:, :64], jnp.uint32)
    bb = lax.bitcast_convert_type(v[:, 64:], jnp.uint32)
    ar = ab + 0x7FFF + ((ab >> 16) & 1)
    br = bb + 0x7FFF + ((bb >> 16) & 1)
    packed = (ar >> 16) | (br & jnp.uint32(0xFFFF0000))
    return lax.bitcast_convert_type(packed, jnp.int32)


def _edge_linear_body(ea_ref, we_ref, be_ref, ee0_ref, ee1_ref):
    ee = (jnp.dot(ea_ref[...], we_ref[...],
                  preferred_element_type=jnp.float32)
          + be_ref[...])
    ee0_ref[...] = _pack_bf16_pair(ee[:, :D])
    ee1_ref[...] = _pack_bf16_pair(ee[:, D:])


def _edge_linear2(edge_attr, We0, be0, We1, be1):
    BE = 6400
    We = jnp.concatenate([We0, We1], axis=1)
    be = jnp.concatenate([be0, be1]).reshape(1, 2 * D)
    return pl.pallas_call(
        _edge_linear_body,
        grid=(E // BE,),
        in_specs=[
            pl.BlockSpec((BE, DE), lambda i: (i, 0)),
            pl.BlockSpec((DE, 2 * D), lambda i: (0, 0)),
            pl.BlockSpec((1, 2 * D), lambda i: (0, 0)),
        ],
        out_specs=[pl.BlockSpec((BE, D // 2), lambda i: (i, 0)),
                   pl.BlockSpec((BE, D // 2), lambda i: (i, 0))],
        out_shape=[jax.ShapeDtypeStruct((E, D // 2), jnp.int32),
                   jax.ShapeDtypeStruct((E, D // 2), jnp.int32)],
    )(edge_attr, We, be)


def _edge_stage_body(x_hbm, ee_hbm, src_hbm, dst_hbm, out_hbm,
                     src_idx, dst_idx, rows, eebuf, agg_sh,
                     sg0, sg1, sg2, sg3, se0, se1, se2, se3,
                     ss0, ss1, ss2, ss3, si0, si1, si2, si3,
                     sd0, sd1, sd2, sd3):
    cid = lax.axis_index("c")
    sid = lax.axis_index("s")
    wid = sid * NC + cid
    base_w = wid * EPW

    sem_g = [sg0, sg1, sg2, sg3]
    sem_e = [se0, se1, se2, se3]
    sem_s = [ss0, ss1, ss2, ss3]
    sem_i = [si0, si1, si2, si3]
    sem_d = [sd0, sd1, sd2, sd3]

    zv = jnp.zeros((16,), jnp.float32)

    @pl.loop(0, G)
    def _zero_fill(r):
        for cc in range(D // 16):
            rows[0, r, pl.ds(cc * 16, 16)] = zv

    @pl.loop(sid, NCH, step=NS)
    def _zero_out(j):
        pltpu.sync_copy(rows.at[0, pl.ds(0, CHO)],
                        agg_sh.at[pl.ds(j * CHO, CHO)])

    plsc.subcore_barrier()

    def issue_src_idx(k, islot):
        base = base_w + k * G
        pltpu.async_copy(src_hbm.at[pl.ds(base, G)], src_idx.at[islot],
                         sem_i[islot])

    def wait_src_idx(islot):
        pltpu.make_async_copy(src_hbm.at[pl.ds(0, G)], src_idx.at[islot],
                              sem_i[islot]).wait()

    def issue_dst_idx(k, islot):
        base = base_w + k * G
        pltpu.async_copy(dst_hbm.at[pl.ds(base, G)], dst_idx.at[islot],
                         sem_d[islot])

    def wait_dst_idx(islot):
        pltpu.make_async_copy(dst_hbm.at[pl.ds(0, G)], dst_idx.at[islot],
                              sem_d[islot]).wait()

    def issue_gather_ee(k, b, islot):
        base = base_w + k * G
        pltpu.async_copy(x_hbm.at[src_idx.at[islot]], rows.at[b], sem_g[b])
        pltpu.async_copy(ee_hbm.at[pl.ds(base, G)], eebuf.at[b], sem_e[b])

    def phase(k, p):
        b = p % 4
        b2 = (p + 2) % 4
        pltpu.make_async_copy(x_hbm.at[pl.ds(0, G), :], rows.at[b],
                              sem_g[b]).wait()
        pltpu.make_async_copy(ee_hbm.at[pl.ds(0, G), :], eebuf.at[b],
                              sem_e[b]).wait()

        @pl.when(k + 2 < KCH)
        def _():
            @pl.when(k >= 2)
            def _():
                pltpu.make_async_copy(rows.at[b2], agg_sh.at[pl.ds(0, G)],
                                      sem_s[b2]).wait()
            issue_dst_idx(k + 2, b2)
            wait_src_idx(b2)
            issue_gather_ee(k + 2, b2, b2)

        @pl.when(k + 3 < KCH)
        def _():
            issue_src_idx(k + 3, (p + 3) % 4)

        @pl.loop(0, G)
        def _relu_add(r):
            for j in range(D // 32):
                c = j * 16
                ev = eebuf[b, r, pl.ds(c, 16)]
                lo = lax.bitcast_convert_type(ev << 16, jnp.float32)
                hi = lax.bitcast_convert_type(ev & jnp.int32(-65536),
                                              jnp.float32)
                v1 = rows[b, r, pl.ds(c, 16)] + lo
                v2 = rows[b, r, pl.ds(64 + c, 16)] + hi
                rows[b, r, pl.ds(c, 16)] = jnp.maximum(v1, 0.0)
                rows[b, r, pl.ds(64 + c, 16)] = jnp.maximum(v2, 0.0)

        wait_dst_idx(b)
        pltpu.async_copy(rows.at[b], agg_sh.at[dst_idx.at[b]], sem_s[b],
                         add=True)

    issue_src_idx(0, 0)
    issue_src_idx(1, 1)
    issue_src_idx(2, 2)
    issue_dst_idx(0, 0)
    issue_dst_idx(1, 1)
    wait_src_idx(0)
    issue_gather_ee(0, 0, 0)
    wait_src_idx(1)
    issue_gather_ee(1, 1, 1)

    @pl.loop(0, KCH - 2, step=4)
    def _main(k0):
        for p in range(4):
            phase(k0 + p, p)

    phase(KCH - 2, (KCH - 2) % 4)
    phase(KCH - 1, (KCH - 1) % 4)

    for q in range(4):
        pltpu.make_async_copy(rows.at[q], agg_sh.at[pl.ds(0, G)],
                              sem_s[q]).wait()

    plsc.subcore_barrier()

    @pl.loop(sid, NCH_OUT, step=NS)
    def _copy_out(j):
        r0 = j * CHO_OUT
        pltpu.sync_copy(agg_sh.at[pl.ds(r0, CHO_OUT)],
                        out_hbm.at[cid, pl.ds(r0, CHO_OUT), :])


def _edge_stage(x, ee, src, dst):
    mesh = plsc.VectorSubcoreMesh(core_axis_name="c", subcore_axis_name="s",
                                  num_cores=NC, num_subcores=NS)
    f = pl.kernel(
        _edge_stage_body,
        out_type=jax.ShapeDtypeStruct((NC, N, D), jnp.float32),
        mesh=mesh,
        compiler_params=pltpu.CompilerParams(needs_layout_passes=False),
        scratch_types=[
            pltpu.VMEM((4, G), jnp.int32),
            pltpu.VMEM((4, G), jnp.int32),
            pltpu.VMEM((4, G, D), jnp.float32),
            pltpu.VMEM((4, G, D // 2), jnp.int32),
            pltpu.VMEM_SHARED((N, D), jnp.float32),
        ] + [pltpu.SemaphoreType.DMA] * 20,
    )
    return f(x, ee, src, dst)


def _node_stage_body(x_ref, agg_ref, eps_ref, w1_ref, b1_ref, w2_ref, b2_ref,
                     g_ref, bt_ref, out_ref):
    x = x_ref[...]
    h = (1.0 + eps_ref[0, 0]) * x + agg_ref[0] + agg_ref[1]
    t = jnp.maximum(jnp.dot(h, w1_ref[...],
                            preferred_element_type=jnp.float32) + b1_ref[...],
                    0.0)
    o = jnp.dot(t, w2_ref[...],
                preferred_element_type=jnp.float32) + b2_ref[...]
    mu = jnp.mean(o, axis=-1, keepdims=True)
    var = jnp.mean(jnp.square(o - mu), axis=-1, keepdims=True)
    o = (o - mu) * lax.rsqrt(var + 1e-5) * g_ref[...] + bt_ref[...]
    out_ref[...] = jnp.maximum(o, 0.0)


def _node_stage(x, agg, eps, W1, b1, W2, b2, g, bt):
    BN = 2000
    grid = (N // BN,)
    return pl.pallas_call(
        _node_stage_body,
        grid=grid,
        in_specs=[
            pl.BlockSpec((BN, D), lambda i: (i, 0)),
            pl.BlockSpec((NC, BN, D), lambda i: (0, i, 0)),
            pl.BlockSpec((1, 1), lambda i: (0, 0)),
            pl.BlockSpec((D, D), lambda i: (0, 0)),
            pl.BlockSpec((1, D), lambda i: (0, 0)),
            pl.BlockSpec((D, D), lambda i: (0, 0)),
            pl.BlockSpec((1, D), lambda i: (0, 0)),
            pl.BlockSpec((1, D), lambda i: (0, 0)),
            pl.BlockSpec((1, D), lambda i: (0, 0)),
        ],
        out_specs=pl.BlockSpec((BN, D), lambda i: (i, 0)),
        out_shape=jax.ShapeDtypeStruct((N, D), jnp.float32),
    )(x, agg, eps.reshape(1, 1), W1, b1.reshape(1, D), W2, b2.reshape(1, D),
      g.reshape(1, D), bt.reshape(1, D))


def kernel(x, edge_index, edge_attr,
           We0, be0, W10, b10, W20, b20, eps0, g0, bt0,
           We1, be1, W11, b11, W21, b21, eps1, g1, bt1):
    src = edge_index[0].astype(jnp.int32)
    dst = edge_index[1].astype(jnp.int32)

    ee0, ee1 = _edge_linear2(edge_attr, We0, be0, We1, be1)

    agg0 = _edge_stage(x, ee0, src, dst)
    h1 = _node_stage(x, agg0, eps0, W10, b10, W20, b20, g0, bt0)

    agg1 = _edge_stage(h1, ee1, src, dst)
    h2 = _node_stage(h1, agg1, eps1, W11, b11, W21, b21, g1, bt1)
    return h2
